# Initial kernel scaffold; baseline (speedup 1.0000x reference)
import jax
import jax.numpy as jnp
from jax.experimental import pallas as pl


def kernel(indices, mu, log_var, raw_alpha, features):
    raise NotImplementedError("write your pallas kernel here")


# speedup vs baseline: 2.0165x; 2.0165x over previous
import functools

import jax
import jax.numpy as jnp
from jax import lax
from jax.experimental import pallas as pl
from jax.experimental.pallas import tpu as pltpu
from jax.experimental.pallas import tpu_sc as plsc

D_S = 64
D_F = 300
WFA = 304
C = 128


def _build(num_rows):
    info = plsc.get_sparse_core_info()
    nc, ns, nl = info.num_cores, info.num_subcores, info.num_lanes
    nw = nc * ns
    assert num_rows % (nw * C) == 0
    cpw = num_rows // (nw * C)

    mesh = plsc.VectorSubcoreMesh(core_axis_name="c", subcore_axis_name="s")

    @functools.partial(
        pl.kernel,
        mesh=mesh,
        compiler_params=pltpu.CompilerParams(use_tc_tiling_on_sc=False,
                                             needs_layout_passes=False),
        out_type=[
            jax.ShapeDtypeStruct((num_rows, D_S), jnp.float32),
            jax.ShapeDtypeStruct((num_rows, D_S), jnp.float32),
            jax.ShapeDtypeStruct((num_rows,), jnp.float32),
            jax.ShapeDtypeStruct((num_rows, WFA), jnp.float32),
        ],
        scratch_types=[
            pltpu.VMEM((1, cpw, C), jnp.int32),
            pltpu.VMEM((C, D_S), jnp.float32),
            pltpu.VMEM((C, D_S), jnp.float32),
            pltpu.VMEM((C, WFA), jnp.float32),
            pltpu.VMEM((C,), jnp.float32),
            pltpu.SemaphoreType.DMA,
        ],
    )
    def gather_kernel(idx_hbm, mu_hbm, lv_hbm, fa_hbm,
                      mu_o, lv_o, al_o, feat_o,
                      idx_v, mu_v, lv_v, fa_v, al_v, sem):
        wid = lax.axis_index("s") * nc + lax.axis_index("c")
        crow = wid * cpw
        pltpu.sync_copy(idx_hbm.at[pl.ds(wid, 1)], idx_v)

        def chunk(j, carry):
            base = (crow + j) * C
            idx_row = idx_v.at[0, j]
            cp_mu = pltpu.async_copy(mu_hbm.at[idx_row], mu_v, sem)
            cp_lv = pltpu.async_copy(lv_hbm.at[idx_row], lv_v, sem)
            cp_fa = pltpu.async_copy(fa_hbm.at[idx_row], fa_v, sem)
            cp_mu.wait()
            cp_lv.wait()
            cp_fa.wait()
            cols = jnp.full((nl,), D_F, dtype=jnp.int32)
            for i in range(C // nl):
                rows = lax.broadcasted_iota(jnp.int32, (nl,), 0) + i * nl
                v = plsc.load_gather(fa_v, [rows, cols])
                al_v[pl.ds(i * nl, nl)] = 1.0 / (1.0 + jnp.exp(-v))
            pltpu.sync_copy(mu_v, mu_o.at[pl.ds(base, C)])
            pltpu.sync_copy(lv_v, lv_o.at[pl.ds(base, C)])
            pltpu.sync_copy(al_v, al_o.at[pl.ds(base, C)])
            pltpu.sync_copy(fa_v, feat_o.at[pl.ds(base, C)])
            return carry

        lax.fori_loop(0, cpw, chunk, 0)

    return gather_kernel


def kernel(indices, mu, log_var, raw_alpha, features):
    b, s = indices.shape
    n = b * s
    v = mu.shape[0]
    info = plsc.get_sparse_core_info()
    nw = info.num_cores * info.num_subcores
    idx = indices.astype(jnp.int32).reshape(nw, n // (nw * C), C)
    fa = jnp.concatenate(
        [features, raw_alpha[:, None], jnp.zeros((v, WFA - D_F - 1), jnp.float32)],
        axis=1)
    gk = _build(n)
    mu_o, lv_o, al_o, feat_o = gk(idx, mu, log_var, fa)
    return (mu_o.reshape(b, s, D_S), lv_o.reshape(b, s, D_S),
            al_o.reshape(b, s), feat_o[:, :D_F].reshape(b, s, D_F))
